# baseline (speedup 1.0000x reference)
import jax
import jax.numpy as jnp
from jax import lax
from jax.experimental import pallas as pl
from jax.experimental.pallas import tpu as pltpu
from jax.experimental.pallas import tpu_sc as plsc

T = 4096
IN_DIM = 512
HIDDEN_DIM = 64
K = 32
BLOCK_R = 256
NBLK = T // BLOCK_R

NPARTS = 4
PT = T // NPARTS
NWORKERS = 32
ROWS_PER_W = PT // NWORKERS
BATCH = 8
NBATCH = ROWS_PER_W // BATCH
CAP = 128
NEG = -3.0e38


def _h_kernel(x_ref, w_ref, b_ref, h_ref):
    h = lax.dot_general(x_ref[...], w_ref[...], (((1,), (1,)), ((), ())),
                        preferred_element_type=jnp.float32)
    h_ref[...] = h + b_ref[...]


def _main_kernel(hb_ref, hf_ref, adj_ref, thr_ref):
    sim = lax.dot_general(hb_ref[...], hf_ref[...], (((1,), (1,)), ((), ())),
                          preferred_element_type=jnp.float32)
    m0 = jnp.max(sim, axis=1, keepdims=True)
    p = jnp.exp(sim - m0)
    s = jnp.sum(p, axis=1, keepdims=True)
    a = p / s
    adj_ref[...] = a

    vm = a[:, 0:256]
    for c in range(1, 16):
        vm = jnp.maximum(vm, a[:, c * 256:(c + 1) * 256])

    for _ in range(K - 1):
        m = jnp.max(vm, axis=1, keepdims=True)
        vm = jnp.where(vm == m, NEG, vm)
    thr_ref[...] = jnp.broadcast_to(
        jnp.max(vm, axis=1, keepdims=True), (BLOCK_R, 16))


def _merge16(av, ai, bv, bi):
    rbv = lax.rev(bv, (0,))
    rbi = lax.rev(bi, (0,))
    m = av >= rbv
    hv = jnp.where(m, av, rbv)
    hi = jnp.where(m, ai, rbi)
    lv = jnp.where(m, rbv, av)
    li = jnp.where(m, rbi, ai)
    hv, hi = plsc.sort_key_val(hv, hi, descending=True)
    lv, li = plsc.sort_key_val(lv, li, descending=True)
    return hv, hi, lv, li


def _merge32_top(a, b):
    ahv, ahi, alv, ali = a
    bhv, bhi, blv, bli = b
    rblv, rbli = lax.rev(blv, (0,)), lax.rev(bli, (0,))
    rbhv, rbhi = lax.rev(bhv, (0,)), lax.rev(bhi, (0,))
    m1 = ahv >= rblv
    t1v = jnp.where(m1, ahv, rblv)
    t1i = jnp.where(m1, ahi, rbli)
    m2 = alv >= rbhv
    t2v = jnp.where(m2, alv, rbhv)
    t2i = jnp.where(m2, ali, rbhi)
    m = t1v >= t2v
    u1v = jnp.where(m, t1v, t2v)
    u1i = jnp.where(m, t1i, t2i)
    u2v = jnp.where(m, t2v, t1v)
    u2i = jnp.where(m, t2i, t1i)
    u1v, u1i = plsc.sort_key_val(u1v, u1i, descending=True)
    u2v, u2i = plsc.sort_key_val(u2v, u2i, descending=True)
    return u1v, u1i, u2v, u2i


def _process_row(rowbuf, base_off, tsplat, valbuf, idxbuf, stage, stage_off):
    neg = jnp.full((16,), NEG, jnp.float32)
    zero16 = jnp.zeros((16,), jnp.int32)
    for q in range(CAP // 16):
        valbuf[pl.ds(q * 16, 16)] = neg
        idxbuf[pl.ds(q * 16, 16)] = zero16

    lane = lax.iota(jnp.int32, 16)

    def scan_body(jj, off):
        sorted_chunks = []
        for u in range(16):
            j = jj * 16 + u
            v = rowbuf[pl.ds(base_off + j * 16, 16)]
            msk = v >= tsplat
            sv, si = plsc.sort_key_val(jnp.where(msk, v, NEG),
                                       lane + j * 16, descending=True)
            cnt = plsc.all_reduce_population_count(msk)[0]
            sorted_chunks.append((sv, si, cnt))
        for sv, si, cnt in sorted_chunks:
            valbuf[pl.ds(off, 16)] = sv
            idxbuf[pl.ds(off, 16)] = si
            off = jnp.minimum(off + cnt, CAP - 16)
        return off

    off_final = lax.fori_loop(0, T // 256, scan_body, jnp.int32(0))

    def select(nlists):
        lists = []
        for q in range(nlists):
            v = valbuf[pl.ds(q * 16, 16)]
            i = idxbuf[pl.ds(q * 16, 16)]
            lists.append(plsc.sort_key_val(v, i, descending=True))
        s32 = [_merge16(lists[2 * j][0], lists[2 * j][1],
                        lists[2 * j + 1][0], lists[2 * j + 1][1])
               for j in range(nlists // 2)]
        while len(s32) > 1:
            s32 = [_merge32_top(s32[2 * j], s32[2 * j + 1])
                   for j in range(len(s32) // 2)]
        top = s32[0]
        stage[pl.ds(stage_off, 16)] = top[1]
        stage[pl.ds(stage_off + 16, 16)] = top[3]

    del off_final
    select(CAP // 16)


def _topk_sc(adj_hbm, thr_hbm, out_hbm, buf0, buf1, thrbuf, valbuf, idxbuf,
             stage, sem0, sem1):
    nc = 2
    wid = lax.axis_index("s") * nc + lax.axis_index("c")
    base = wid * ROWS_PER_W

    pltpu.sync_copy(thr_hbm.at[pl.ds(base * 16, ROWS_PER_W * 16)], thrbuf)

    def fetch(bi, buf, sem):
        first = base + bi * BATCH
        for r in range(BATCH):
            pltpu.async_copy(
                adj_hbm.at[first + r], buf.at[pl.ds(r * T, T)], sem)

    fetch(0, buf0, sem0)
    fetch(1, buf1, sem1)

    def full_body(bi, carry):
        even = bi % 2 == 0

        def wait(buf, sem):
            for r in range(BATCH):
                pltpu.make_async_copy(
                    adj_hbm.at[0], buf.at[pl.ds(r * T, T)], sem).wait()

        def rows(buf):
            for r in range(BATCH):
                row = bi * BATCH + r
                tsplat = thrbuf[pl.ds(row * 16, 16)]
                _process_row(buf, r * T, tsplat, valbuf, idxbuf,
                             stage, row * K)

        nxt = jnp.minimum(bi + 2, NBATCH - 1)

        def prefetch(buf, sem):
            fetch(nxt, buf, sem)

        @pl.when(even)
        def _():
            wait(buf0, sem0)
            rows(buf0)
            prefetch(buf0, sem0)

        @pl.when(jnp.logical_not(even))
        def _():
            wait(buf1, sem1)
            rows(buf1)
            prefetch(buf1, sem1)

        return carry

    lax.fori_loop(0, NBATCH, full_body, jnp.int32(0))

    for r in range(BATCH):
        pltpu.make_async_copy(
            adj_hbm.at[0], buf0.at[pl.ds(r * T, T)], sem0).wait()
        pltpu.make_async_copy(
            adj_hbm.at[0], buf1.at[pl.ds(r * T, T)], sem1).wait()

    pltpu.sync_copy(stage, out_hbm.at[pl.ds(base * K, ROWS_PER_W * K)])


def kernel(x, W, b):
    h = pl.pallas_call(
        _h_kernel,
        out_shape=jax.ShapeDtypeStruct((T, HIDDEN_DIM), jnp.float32),
    )(x, W, b.reshape(1, HIDDEN_DIM))

    mesh = plsc.VectorSubcoreMesh(core_axis_name="c", subcore_axis_name="s")
    sc_topk = pl.kernel(
        _topk_sc,
        mesh=mesh,
        compiler_params=pltpu.CompilerParams(needs_layout_passes=False),
        out_type=jax.ShapeDtypeStruct((PT * K,), jnp.int32),
        scratch_types=[
            pltpu.VMEM((BATCH * T,), jnp.float32),
            pltpu.VMEM((BATCH * T,), jnp.float32),
            pltpu.VMEM((ROWS_PER_W * 16,), jnp.float32),
            pltpu.VMEM((CAP,), jnp.float32),
            pltpu.VMEM((CAP,), jnp.int32),
            pltpu.VMEM((ROWS_PER_W * K,), jnp.int32),
            pltpu.SemaphoreType.DMA,
            pltpu.SemaphoreType.DMA,
        ],
    )

    adj_parts = []
    topk_parts = []
    for p in range(NPARTS):
        adj_p, thr_p = pl.pallas_call(
            _main_kernel,
            grid=(PT // BLOCK_R,),
            in_specs=[
                pl.BlockSpec((BLOCK_R, HIDDEN_DIM),
                             lambda i, p=p: (i + p * (PT // BLOCK_R), 0)),
                pl.BlockSpec((T, HIDDEN_DIM), lambda i: (0, 0)),
            ],
            out_specs=[
                pl.BlockSpec((BLOCK_R, T), lambda i: (i, 0)),
                pl.BlockSpec((BLOCK_R, 16), lambda i: (i, 0)),
            ],
            out_shape=[
                jax.ShapeDtypeStruct((PT, T), jnp.float32),
                jax.ShapeDtypeStruct((PT, 16), jnp.float32),
            ],
        )(h, h)
        adj_parts.append(adj_p)
        topk_parts.append(sc_topk(adj_p, thr_p.reshape(PT * 16)))

    adj = jnp.concatenate(adj_parts, axis=0)
    topk_flat = jnp.concatenate(topk_parts, axis=0)
    row = jnp.repeat(jnp.arange(T, dtype=jnp.int32), K)
    edge_index = jnp.stack([row, topk_flat], axis=0)
    return adj, edge_index


# speedup vs baseline: 1.1386x; 1.1386x over previous
import jax
import jax.numpy as jnp
from jax import lax
from jax.experimental import pallas as pl
from jax.experimental.pallas import tpu as pltpu
from jax.experimental.pallas import tpu_sc as plsc

T = 4096
IN_DIM = 512
HIDDEN_DIM = 64
K = 32
BLOCK_R = 256
NBLK = T // BLOCK_R

NPARTS = 2
PT = T // NPARTS
NWORKERS = 32
ROWS_PER_W = PT // NWORKERS
BATCH = 8
NBATCH = ROWS_PER_W // BATCH
CAP = 128
NEG = -3.0e38


def _h_kernel(x_ref, w_ref, b_ref, h_ref):
    h = lax.dot_general(x_ref[...], w_ref[...], (((1,), (1,)), ((), ())),
                        preferred_element_type=jnp.float32)
    h_ref[...] = h + b_ref[...]


def _main_kernel(hb_ref, hf_ref, adj_ref, thr_ref):
    sim = lax.dot_general(hb_ref[...], hf_ref[...], (((1,), (1,)), ((), ())),
                          preferred_element_type=jnp.float32)
    m0 = jnp.max(sim, axis=1, keepdims=True)
    p = jnp.exp(sim - m0)
    s = jnp.sum(p, axis=1, keepdims=True)
    a = p / s
    adj_ref[...] = a

    vm = a[:, 0:256]
    for c in range(1, 16):
        vm = jnp.maximum(vm, a[:, c * 256:(c + 1) * 256])

    for _ in range(K - 1):
        m = jnp.max(vm, axis=1, keepdims=True)
        vm = jnp.where(vm == m, NEG, vm)
    thr_ref[...] = jnp.broadcast_to(
        jnp.max(vm, axis=1, keepdims=True), (BLOCK_R, 16))


def _merge16(av, ai, bv, bi):
    rbv = lax.rev(bv, (0,))
    rbi = lax.rev(bi, (0,))
    m = av >= rbv
    hv = jnp.where(m, av, rbv)
    hi = jnp.where(m, ai, rbi)
    lv = jnp.where(m, rbv, av)
    li = jnp.where(m, rbi, ai)
    hv, hi = plsc.sort_key_val(hv, hi, descending=True)
    lv, li = plsc.sort_key_val(lv, li, descending=True)
    return hv, hi, lv, li


def _merge32_top(a, b):
    ahv, ahi, alv, ali = a
    bhv, bhi, blv, bli = b
    rblv, rbli = lax.rev(blv, (0,)), lax.rev(bli, (0,))
    rbhv, rbhi = lax.rev(bhv, (0,)), lax.rev(bhi, (0,))
    m1 = ahv >= rblv
    t1v = jnp.where(m1, ahv, rblv)
    t1i = jnp.where(m1, ahi, rbli)
    m2 = alv >= rbhv
    t2v = jnp.where(m2, alv, rbhv)
    t2i = jnp.where(m2, ali, rbhi)
    m = t1v >= t2v
    u1v = jnp.where(m, t1v, t2v)
    u1i = jnp.where(m, t1i, t2i)
    u2v = jnp.where(m, t2v, t1v)
    u2i = jnp.where(m, t2i, t1i)
    u1v, u1i = plsc.sort_key_val(u1v, u1i, descending=True)
    u2v, u2i = plsc.sort_key_val(u2v, u2i, descending=True)
    return u1v, u1i, u2v, u2i


def _process_row(rowbuf, base_off, tsplat, valbuf, idxbuf, stage, stage_off):
    neg = jnp.full((16,), NEG, jnp.float32)
    zero16 = jnp.zeros((16,), jnp.int32)
    for q in range(CAP // 16):
        valbuf[pl.ds(q * 16, 16)] = neg
        idxbuf[pl.ds(q * 16, 16)] = zero16

    lane = lax.iota(jnp.int32, 16)

    def scan_body(jj, off):
        sorted_chunks = []
        for u in range(32):
            j = jj * 32 + u
            v = rowbuf[pl.ds(base_off + j * 16, 16)]
            msk = v >= tsplat
            sv, si = plsc.sort_key_val(jnp.where(msk, v, NEG),
                                       lane + j * 16, descending=True)
            cnt = plsc.all_reduce_population_count(msk)[0]
            sorted_chunks.append((sv, si, cnt))
        for sv, si, cnt in sorted_chunks:
            valbuf[pl.ds(off, 16)] = sv
            idxbuf[pl.ds(off, 16)] = si
            off = jnp.minimum(off + cnt, CAP - 16)
        return off

    off_final = lax.fori_loop(0, T // 512, scan_body, jnp.int32(0))

    def select(nlists):
        lists = []
        for q in range(nlists):
            v = valbuf[pl.ds(q * 16, 16)]
            i = idxbuf[pl.ds(q * 16, 16)]
            lists.append(plsc.sort_key_val(v, i, descending=True))
        s32 = [_merge16(lists[2 * j][0], lists[2 * j][1],
                        lists[2 * j + 1][0], lists[2 * j + 1][1])
               for j in range(nlists // 2)]
        while len(s32) > 1:
            s32 = [_merge32_top(s32[2 * j], s32[2 * j + 1])
                   for j in range(len(s32) // 2)]
        top = s32[0]
        stage[pl.ds(stage_off, 16)] = top[1]
        stage[pl.ds(stage_off + 16, 16)] = top[3]

    del off_final
    select(CAP // 16)


def _topk_sc(adj_hbm, thr_hbm, out_hbm, buf0, buf1, thrbuf, valbuf, idxbuf,
             stage, sem0, sem1):
    nc = 2
    wid = lax.axis_index("s") * nc + lax.axis_index("c")
    base = wid * ROWS_PER_W

    pltpu.sync_copy(thr_hbm.at[pl.ds(base * 16, ROWS_PER_W * 16)], thrbuf)

    def fetch(bi, buf, sem):
        first = base + bi * BATCH
        for r in range(BATCH):
            pltpu.async_copy(
                adj_hbm.at[first + r], buf.at[pl.ds(r * T, T)], sem)

    fetch(0, buf0, sem0)
    fetch(1, buf1, sem1)

    def full_body(bi, carry):
        even = bi % 2 == 0

        def wait(buf, sem):
            for r in range(BATCH):
                pltpu.make_async_copy(
                    adj_hbm.at[0], buf.at[pl.ds(r * T, T)], sem).wait()

        def rows(buf):
            for r in range(BATCH):
                row = bi * BATCH + r
                tsplat = thrbuf[pl.ds(row * 16, 16)]
                _process_row(buf, r * T, tsplat, valbuf, idxbuf,
                             stage, row * K)

        nxt = jnp.minimum(bi + 2, NBATCH - 1)

        def prefetch(buf, sem):
            fetch(nxt, buf, sem)

        @pl.when(even)
        def _():
            wait(buf0, sem0)
            rows(buf0)
            prefetch(buf0, sem0)

        @pl.when(jnp.logical_not(even))
        def _():
            wait(buf1, sem1)
            rows(buf1)
            prefetch(buf1, sem1)

        return carry

    lax.fori_loop(0, NBATCH, full_body, jnp.int32(0))

    for r in range(BATCH):
        pltpu.make_async_copy(
            adj_hbm.at[0], buf0.at[pl.ds(r * T, T)], sem0).wait()
        pltpu.make_async_copy(
            adj_hbm.at[0], buf1.at[pl.ds(r * T, T)], sem1).wait()

    pltpu.sync_copy(stage, out_hbm.at[pl.ds(base * K, ROWS_PER_W * K)])


def kernel(x, W, b):
    h = pl.pallas_call(
        _h_kernel,
        out_shape=jax.ShapeDtypeStruct((T, HIDDEN_DIM), jnp.float32),
    )(x, W, b.reshape(1, HIDDEN_DIM))

    mesh = plsc.VectorSubcoreMesh(core_axis_name="c", subcore_axis_name="s")
    sc_topk = pl.kernel(
        _topk_sc,
        mesh=mesh,
        compiler_params=pltpu.CompilerParams(needs_layout_passes=False),
        out_type=jax.ShapeDtypeStruct((PT * K,), jnp.int32),
        scratch_types=[
            pltpu.VMEM((BATCH * T,), jnp.float32),
            pltpu.VMEM((BATCH * T,), jnp.float32),
            pltpu.VMEM((ROWS_PER_W * 16,), jnp.float32),
            pltpu.VMEM((CAP,), jnp.float32),
            pltpu.VMEM((CAP,), jnp.int32),
            pltpu.VMEM((ROWS_PER_W * K,), jnp.int32),
            pltpu.SemaphoreType.DMA,
            pltpu.SemaphoreType.DMA,
        ],
    )

    adj_parts = []
    topk_parts = []
    for p in range(NPARTS):
        adj_p, thr_p = pl.pallas_call(
            _main_kernel,
            grid=(PT // BLOCK_R,),
            in_specs=[
                pl.BlockSpec((BLOCK_R, HIDDEN_DIM),
                             lambda i, p=p: (i + p * (PT // BLOCK_R), 0)),
                pl.BlockSpec((T, HIDDEN_DIM), lambda i: (0, 0)),
            ],
            out_specs=[
                pl.BlockSpec((BLOCK_R, T), lambda i: (i, 0)),
                pl.BlockSpec((BLOCK_R, 16), lambda i: (i, 0)),
            ],
            out_shape=[
                jax.ShapeDtypeStruct((PT, T), jnp.float32),
                jax.ShapeDtypeStruct((PT, 16), jnp.float32),
            ],
        )(h, h)
        adj_parts.append(adj_p)
        topk_parts.append(sc_topk(adj_p, thr_p.reshape(PT * 16)))

    adj = jnp.concatenate(adj_parts, axis=0)
    topk_flat = jnp.concatenate(topk_parts, axis=0)
    row = jnp.repeat(jnp.arange(T, dtype=jnp.int32), K)
    edge_index = jnp.stack([row, topk_flat], axis=0)
    return adj, edge_index


# speedup vs baseline: 1.2296x; 1.0799x over previous
import jax
import jax.numpy as jnp
from jax import lax
from jax.experimental import pallas as pl
from jax.experimental.pallas import tpu as pltpu
from jax.experimental.pallas import tpu_sc as plsc

T = 4096
IN_DIM = 512
HIDDEN_DIM = 64
K = 32
BLOCK_R = 512
NBLK = T // BLOCK_R

NPARTS = 2
PT = T // NPARTS
NWORKERS = 32
ROWS_PER_W = PT // NWORKERS
BATCH = 8
NBATCH = ROWS_PER_W // BATCH
CAP = 128
NEG = -3.0e38


def _h_kernel(x_ref, w_ref, b_ref, h_ref):
    h = lax.dot_general(x_ref[...], w_ref[...], (((1,), (1,)), ((), ())),
                        preferred_element_type=jnp.float32)
    h_ref[...] = h + b_ref[...]


def _main_kernel(hb_ref, hf_ref, adj_ref, thr_ref):
    sim = lax.dot_general(hb_ref[...], hf_ref[...], (((1,), (1,)), ((), ())),
                          preferred_element_type=jnp.float32)
    m0 = jnp.max(sim, axis=1, keepdims=True)
    p = jnp.exp(sim - m0)
    s = jnp.sum(p, axis=1, keepdims=True)
    a = p * (1.0 / s)
    adj_ref[...] = a

    vm = a[:, 0:256]
    for c in range(1, 16):
        vm = jnp.maximum(vm, a[:, c * 256:(c + 1) * 256])

    for _ in range(K - 1):
        m = jnp.max(vm, axis=1, keepdims=True)
        vm = jnp.where(vm == m, NEG, vm)
    thr_ref[...] = jnp.broadcast_to(
        jnp.max(vm, axis=1, keepdims=True), (BLOCK_R, 16))


def _merge16(av, ai, bv, bi):
    rbv = lax.rev(bv, (0,))
    rbi = lax.rev(bi, (0,))
    m = av >= rbv
    hv = jnp.where(m, av, rbv)
    hi = jnp.where(m, ai, rbi)
    lv = jnp.where(m, rbv, av)
    li = jnp.where(m, rbi, ai)
    hv, hi = plsc.sort_key_val(hv, hi, descending=True)
    lv, li = plsc.sort_key_val(lv, li, descending=True)
    return hv, hi, lv, li


def _merge32_top(a, b):
    ahv, ahi, alv, ali = a
    bhv, bhi, blv, bli = b
    rblv, rbli = lax.rev(blv, (0,)), lax.rev(bli, (0,))
    rbhv, rbhi = lax.rev(bhv, (0,)), lax.rev(bhi, (0,))
    m1 = ahv >= rblv
    t1v = jnp.where(m1, ahv, rblv)
    t1i = jnp.where(m1, ahi, rbli)
    m2 = alv >= rbhv
    t2v = jnp.where(m2, alv, rbhv)
    t2i = jnp.where(m2, ali, rbhi)
    m = t1v >= t2v
    u1v = jnp.where(m, t1v, t2v)
    u1i = jnp.where(m, t1i, t2i)
    u2v = jnp.where(m, t2v, t1v)
    u2i = jnp.where(m, t2i, t1i)
    u1v, u1i = plsc.sort_key_val(u1v, u1i, descending=True)
    u2v, u2i = plsc.sort_key_val(u2v, u2i, descending=True)
    return u1v, u1i, u2v, u2i


def _process_row(rowbuf, base_off, tsplat, valbuf, idxbuf, stage, stage_off):
    neg = jnp.full((16,), NEG, jnp.float32)
    zero16 = jnp.zeros((16,), jnp.int32)
    for q in range(CAP // 16):
        valbuf[pl.ds(q * 16, 16)] = neg
        idxbuf[pl.ds(q * 16, 16)] = zero16

    lane = lax.iota(jnp.int32, 16)

    def scan_body(jj, off):
        sorted_chunks = []
        for u in range(32):
            j = jj * 32 + u
            v = rowbuf[pl.ds(base_off + j * 16, 16)]
            msk = v >= tsplat
            sv, si = plsc.sort_key_val(jnp.where(msk, v, NEG),
                                       lane + j * 16, descending=True)
            cnt = plsc.all_reduce_population_count(msk)[0]
            sorted_chunks.append((sv, si, cnt))
        for sv, si, cnt in sorted_chunks:
            valbuf[pl.ds(off, 16)] = sv
            idxbuf[pl.ds(off, 16)] = si
            off = jnp.minimum(off + cnt, CAP - 16)
        return off

    off_final = lax.fori_loop(0, T // 512, scan_body, jnp.int32(0))

    def select(nlists):
        lists = []
        for q in range(nlists):
            v = valbuf[pl.ds(q * 16, 16)]
            i = idxbuf[pl.ds(q * 16, 16)]
            lists.append(plsc.sort_key_val(v, i, descending=True))
        s32 = [_merge16(lists[2 * j][0], lists[2 * j][1],
                        lists[2 * j + 1][0], lists[2 * j + 1][1])
               for j in range(nlists // 2)]
        while len(s32) > 1:
            s32 = [_merge32_top(s32[2 * j], s32[2 * j + 1])
                   for j in range(len(s32) // 2)]
        top = s32[0]
        stage[pl.ds(stage_off, 16)] = top[1]
        stage[pl.ds(stage_off + 16, 16)] = top[3]

    del off_final
    select(CAP // 16)


def _topk_sc(adj_hbm, thr_hbm, out_hbm, buf0, buf1, thrbuf, valbuf, idxbuf,
             stage, sem0, sem1):
    nc = 2
    wid = lax.axis_index("s") * nc + lax.axis_index("c")
    base = wid * ROWS_PER_W

    pltpu.sync_copy(thr_hbm.at[pl.ds(base * 16, ROWS_PER_W * 16)], thrbuf)

    def fetch(bi, buf, sem):
        first = base + bi * BATCH
        for r in range(BATCH):
            pltpu.async_copy(
                adj_hbm.at[first + r], buf.at[pl.ds(r * T, T)], sem)

    fetch(0, buf0, sem0)
    fetch(1, buf1, sem1)

    def full_body(bi, carry):
        even = bi % 2 == 0

        def wait(buf, sem):
            for r in range(BATCH):
                pltpu.make_async_copy(
                    adj_hbm.at[0], buf.at[pl.ds(r * T, T)], sem).wait()

        def rows(buf):
            for r in range(BATCH):
                row = bi * BATCH + r
                tsplat = thrbuf[pl.ds(row * 16, 16)]
                _process_row(buf, r * T, tsplat, valbuf, idxbuf,
                             stage, row * K)

        nxt = jnp.minimum(bi + 2, NBATCH - 1)

        def prefetch(buf, sem):
            fetch(nxt, buf, sem)

        @pl.when(even)
        def _():
            wait(buf0, sem0)
            rows(buf0)
            prefetch(buf0, sem0)

        @pl.when(jnp.logical_not(even))
        def _():
            wait(buf1, sem1)
            rows(buf1)
            prefetch(buf1, sem1)

        return carry

    lax.fori_loop(0, NBATCH, full_body, jnp.int32(0))

    for r in range(BATCH):
        pltpu.make_async_copy(
            adj_hbm.at[0], buf0.at[pl.ds(r * T, T)], sem0).wait()
        pltpu.make_async_copy(
            adj_hbm.at[0], buf1.at[pl.ds(r * T, T)], sem1).wait()

    pltpu.sync_copy(stage, out_hbm.at[pl.ds(base * K, ROWS_PER_W * K)])


def kernel(x, W, b):
    h = pl.pallas_call(
        _h_kernel,
        out_shape=jax.ShapeDtypeStruct((T, HIDDEN_DIM), jnp.float32),
    )(x, W, b.reshape(1, HIDDEN_DIM))

    mesh = plsc.VectorSubcoreMesh(core_axis_name="c", subcore_axis_name="s")
    sc_topk = pl.kernel(
        _topk_sc,
        mesh=mesh,
        compiler_params=pltpu.CompilerParams(needs_layout_passes=False),
        out_type=jax.ShapeDtypeStruct((PT * K,), jnp.int32),
        scratch_types=[
            pltpu.VMEM((BATCH * T,), jnp.float32),
            pltpu.VMEM((BATCH * T,), jnp.float32),
            pltpu.VMEM((ROWS_PER_W * 16,), jnp.float32),
            pltpu.VMEM((CAP,), jnp.float32),
            pltpu.VMEM((CAP,), jnp.int32),
            pltpu.VMEM((ROWS_PER_W * K,), jnp.int32),
            pltpu.SemaphoreType.DMA,
            pltpu.SemaphoreType.DMA,
        ],
    )

    adj_parts = []
    topk_parts = []
    for p in range(NPARTS):
        adj_p, thr_p = pl.pallas_call(
            _main_kernel,
            grid=(PT // BLOCK_R,),
            in_specs=[
                pl.BlockSpec((BLOCK_R, HIDDEN_DIM),
                             lambda i, p=p: (i + p * (PT // BLOCK_R), 0)),
                pl.BlockSpec((T, HIDDEN_DIM), lambda i: (0, 0)),
            ],
            out_specs=[
                pl.BlockSpec((BLOCK_R, T), lambda i: (i, 0)),
                pl.BlockSpec((BLOCK_R, 16), lambda i: (i, 0)),
            ],
            out_shape=[
                jax.ShapeDtypeStruct((PT, T), jnp.float32),
                jax.ShapeDtypeStruct((PT, 16), jnp.float32),
            ],
        )(h, h)
        adj_parts.append(adj_p)
        topk_parts.append(sc_topk(adj_p, thr_p.reshape(PT * 16)))

    adj = jnp.concatenate(adj_parts, axis=0)
    topk_flat = jnp.concatenate(topk_parts, axis=0)
    row = jnp.repeat(jnp.arange(T, dtype=jnp.int32), K)
    edge_index = jnp.stack([row, topk_flat], axis=0)
    return adj, edge_index
